# baseline (speedup 1.0000x reference)
import functools
import jax
import jax.numpy as jnp
from jax import lax
from jax.experimental import pallas as pl
from jax.experimental.pallas import tpu as pltpu
from jax.experimental.pallas import tpu_sc as plsc

N_BOXES = 20000
NUM_CLASS = 3
N_PRE = 2048
N_POST = 500
THRESH = 0.7
ROWS = N_PRE // 128
OUT_ROWS = 512


def _tree(x, op, axis, size):
    s = 1
    while s < size:
        x = op(x, pltpu.roll(x, s, axis))
        s *= 2
    return x


def _nms_scan_body(planes_ref, out_ref):
    cx = planes_ref[0 * ROWS:1 * ROWS, :]
    cy = planes_ref[1 * ROWS:2 * ROWS, :]
    w = planes_ref[2 * ROWS:3 * ROWS, :]
    h = planes_ref[3 * ROWS:4 * ROWS, :]
    sc = planes_ref[4 * ROWS:5 * ROWS, :]
    lb = planes_ref[5 * ROWS:6 * ROWS, :]

    x1 = cx - 0.5 * w
    y1 = cy - 0.5 * h
    x2 = cx + 0.5 * w
    y2 = cy + 0.5 * h
    area = (x2 - x1) * (y2 - y1)

    row_i = jax.lax.broadcasted_iota(jnp.int32, (ROWS, 128), 0)
    col_i = jax.lax.broadcasted_iota(jnp.int32, (ROWS, 128), 1)
    iota2 = (row_i * 128 + col_i).astype(jnp.float32)
    lane128 = jax.lax.broadcasted_iota(jnp.int32, (1, 128), 1)

    def body(i, sup):
        cand = jnp.where(sup != 0, 2.0 * N_PRE, iota2)
        m = jnp.broadcast_to(jnp.min(cand, axis=1, keepdims=True), (ROWS, 128))
        minb = _tree(m, jnp.minimum, 0, ROWS)
        validv = minb < float(N_PRE)
        sel = ((cand == minb) & validv).astype(jnp.float32)

        def colmax(p):
            return _tree(p * sel, jnp.maximum, 0, ROWS)

        comb = jnp.concatenate(
            [colmax(cx)[0:1], colmax(cy)[1:2], colmax(w)[2:3],
             colmax(h)[3:4], colmax(sc)[4:5], colmax(lb)[5:6],
             jnp.zeros((2, 128), jnp.float32)],
            axis=0,
        )
        combb = jnp.broadcast_to(
            jnp.max(comb, axis=1, keepdims=True), (8, 128))

        cxb = jnp.broadcast_to(combb[0:1, :], (ROWS, 128))
        cyb = jnp.broadcast_to(combb[1:2, :], (ROWS, 128))
        wb = jnp.broadcast_to(combb[2:3, :], (ROWS, 128))
        hb = jnp.broadcast_to(combb[3:4, :], (ROWS, 128))
        x1b = cxb - 0.5 * wb
        y1b = cyb - 0.5 * hb
        x2b = cxb + 0.5 * wb
        y2b = cyb + 0.5 * hb
        area_b = (x2b - x1b) * (y2b - y1b)

        iw = jnp.clip(jnp.minimum(x2, x2b) - jnp.maximum(x1, x1b), 0.0, None)
        ih = jnp.clip(jnp.minimum(y2, y2b) - jnp.maximum(y1, y1b), 0.0, None)
        inter = iw * ih
        iou = inter / (area + area_b - inter + 1e-8)
        supn = sup | ((iou >= THRESH) & validv).astype(jnp.int32)

        vf = jnp.where(validv[0:1, :], 1.0, 0.0)
        shifted = (combb[0:1, :] * (lane128 == 0)
                   + combb[1:2, :] * (lane128 == 1)
                   + combb[2:3, :] * (lane128 == 2)
                   + combb[3:4, :] * (lane128 == 3)
                   + combb[4:5, :] * (lane128 == 4)
                   + (combb[5:6, :] + 1.0) * (lane128 == 5))
        out_ref[pl.ds(i, 1), :] = (shifted * vf)[:, 0:8]
        return supn

    sup0 = jnp.zeros((ROWS, 128), dtype=jnp.int32)
    jax.lax.fori_loop(0, N_POST, body, sup0)


N_PAD = 20480
PER_W = N_PAD // 32
SC_LANES = 16


def _sc_head_body(c0_hbm, c1_hbm, c2_hbm, sco_hbm, lab_hbm, v0, v1, v2, vs, vl):
    wid = lax.axis_index("s") * 2 + lax.axis_index("c")
    base = wid * PER_W
    pltpu.sync_copy(c0_hbm.at[pl.ds(base, PER_W)], v0)
    pltpu.sync_copy(c1_hbm.at[pl.ds(base, PER_W)], v1)
    pltpu.sync_copy(c2_hbm.at[pl.ds(base, PER_W)], v2)

    def step(i, _):
        s0 = v0[pl.ds(i * SC_LANES, SC_LANES)]
        s1 = v1[pl.ds(i * SC_LANES, SC_LANES)]
        s2 = v2[pl.ds(i * SC_LANES, SC_LANES)]
        m = jnp.maximum(jnp.maximum(s0, s1), s2)
        lab = jnp.where(s0 == m, 0.0, jnp.where(s1 == m, 1.0, 2.0))
        vs[pl.ds(i * SC_LANES, SC_LANES)] = m
        vl[pl.ds(i * SC_LANES, SC_LANES)] = lab
        return 0

    lax.fori_loop(0, PER_W // SC_LANES, step, 0)
    pltpu.sync_copy(vs, sco_hbm.at[pl.ds(base, PER_W)])
    pltpu.sync_copy(vl, lab_hbm.at[pl.ds(base, PER_W)])


_sc_head = functools.partial(
    pl.kernel,
    mesh=plsc.VectorSubcoreMesh(core_axis_name="c", subcore_axis_name="s"),
    out_type=[
        jax.ShapeDtypeStruct((N_PAD,), jnp.float32),
        jax.ShapeDtypeStruct((N_PAD,), jnp.float32),
    ],
    scratch_types=[
        pltpu.VMEM((PER_W,), jnp.float32),
        pltpu.VMEM((PER_W,), jnp.float32),
        pltpu.VMEM((PER_W,), jnp.float32),
        pltpu.VMEM((PER_W,), jnp.float32),
        pltpu.VMEM((PER_W,), jnp.float32),
    ],
)(_sc_head_body)


def kernel(boxes, cls_preds):
    cp = jnp.pad(cls_preds, ((0, N_PAD - N_BOXES), (0, 0)))
    cpt = cp.T
    scores_p, labels_p = _sc_head(cpt[0], cpt[1], cpt[2])
    scores = scores_p[:N_BOXES]
    labels = labels_p[:N_BOXES].astype(jnp.int32)
    top_scores, top_idx = jax.lax.top_k(scores, N_PRE)
    b = boxes[top_idx]
    lbl = labels[top_idx].astype(jnp.float32)

    planes = jnp.concatenate(
        [
            b[:, 0].reshape(ROWS, 128),
            b[:, 1].reshape(ROWS, 128),
            b[:, 2].reshape(ROWS, 128),
            b[:, 3].reshape(ROWS, 128),
            top_scores.reshape(ROWS, 128),
            lbl.reshape(ROWS, 128),
        ],
        axis=0,
    )

    out = pl.pallas_call(
        _nms_scan_body,
        out_shape=jax.ShapeDtypeStruct((OUT_ROWS, 8), jnp.float32),
    )(planes)

    rois = out[:N_POST, 0:4]
    roi_scores = out[:N_POST, 4]
    roi_labels = out[:N_POST, 5].astype(jnp.int32)
    return rois, roi_scores, roi_labels


# speedup vs baseline: 1.1674x; 1.1674x over previous
import functools
import jax
import jax.numpy as jnp
from jax import lax
from jax.experimental import pallas as pl
from jax.experimental.pallas import tpu as pltpu
from jax.experimental.pallas import tpu_sc as plsc

N_BOXES = 20000
NUM_CLASS = 3
N_PRE = 2048
N_POST = 500
THRESH = 0.7
ROWS = N_PRE // 128
OUT_ROWS = 512


def _tree(x, op, axis, size):
    s = 1
    while s < size:
        x = op(x, pltpu.roll(x, s, axis))
        s *= 2
    return x


def _nms_scan_body(planes_ref, packed_ref, out_ref):
    cx = planes_ref[0 * ROWS:1 * ROWS, :]
    cy = planes_ref[1 * ROWS:2 * ROWS, :]
    w = planes_ref[2 * ROWS:3 * ROWS, :]
    h = planes_ref[3 * ROWS:4 * ROWS, :]

    x1 = cx - 0.5 * w
    y1 = cy - 0.5 * h
    x2 = cx + 0.5 * w
    y2 = cy + 0.5 * h
    area = (x2 - x1) * (y2 - y1)

    row_i = jax.lax.broadcasted_iota(jnp.int32, (ROWS, 128), 0)
    col_i = jax.lax.broadcasted_iota(jnp.int32, (ROWS, 128), 1)
    iota2 = (row_i * 128 + col_i).astype(jnp.float32)
    lane128 = jax.lax.broadcasted_iota(jnp.int32, (1, 128), 1)

    def body(i, sup):
        cand = jnp.where(sup != 0, 2.0 * N_PRE, iota2)
        m = jnp.broadcast_to(jnp.min(cand, axis=1, keepdims=True), (ROWS, 128))
        minb = _tree(m, jnp.minimum, 0, ROWS)

        idxf = minb[0, 0]
        valid = idxf < float(N_PRE)
        idxc = jnp.minimum(idxf, float(N_PRE - 1)).astype(jnp.int32)
        cxs = packed_ref[0, idxc]
        cys = packed_ref[1, idxc]
        ws = packed_ref[2, idxc]
        hs = packed_ref[3, idxc]
        scs = packed_ref[4, idxc]
        lbs = packed_ref[5, idxc]
        x1s = cxs - 0.5 * ws
        y1s = cys - 0.5 * hs
        x2s = cxs + 0.5 * ws
        y2s = cys + 0.5 * hs
        area_s = (x2s - x1s) * (y2s - y1s)

        iw = jnp.clip(jnp.minimum(x2, x2s) - jnp.maximum(x1, x1s), 0.0, None)
        ih = jnp.clip(jnp.minimum(y2, y2s) - jnp.maximum(y1, y1s), 0.0, None)
        inter = iw * ih
        iou = inter / (area + area_s - inter + 1e-8)
        supn = jnp.where(valid, sup | (iou >= THRESH).astype(jnp.int32), sup)

        vf = jnp.where(valid, 1.0, 0.0)
        row = (jnp.where(lane128 == 0, cxs, 0.0)
               + jnp.where(lane128 == 1, cys, 0.0)
               + jnp.where(lane128 == 2, ws, 0.0)
               + jnp.where(lane128 == 3, hs, 0.0)
               + jnp.where(lane128 == 4, scs, 0.0)
               + jnp.where(lane128 == 5, lbs + 1.0, 0.0)) * vf
        out_ref[pl.ds(i, 1), :] = row[:, 0:8]
        return supn

    sup0 = jnp.zeros((ROWS, 128), dtype=jnp.int32)
    jax.lax.fori_loop(0, N_POST, body, sup0)


N_PAD = 20480
PER_W = N_PAD // 32
SC_LANES = 16


def _sc_head_body(c0_hbm, c1_hbm, c2_hbm, sco_hbm, lab_hbm, v0, v1, v2, vs, vl):
    wid = lax.axis_index("s") * 2 + lax.axis_index("c")
    base = wid * PER_W
    pltpu.sync_copy(c0_hbm.at[pl.ds(base, PER_W)], v0)
    pltpu.sync_copy(c1_hbm.at[pl.ds(base, PER_W)], v1)
    pltpu.sync_copy(c2_hbm.at[pl.ds(base, PER_W)], v2)

    def step(i, _):
        s0 = v0[pl.ds(i * SC_LANES, SC_LANES)]
        s1 = v1[pl.ds(i * SC_LANES, SC_LANES)]
        s2 = v2[pl.ds(i * SC_LANES, SC_LANES)]
        m = jnp.maximum(jnp.maximum(s0, s1), s2)
        lab = jnp.where(s0 == m, 0.0, jnp.where(s1 == m, 1.0, 2.0))
        vs[pl.ds(i * SC_LANES, SC_LANES)] = m
        vl[pl.ds(i * SC_LANES, SC_LANES)] = lab
        return 0

    lax.fori_loop(0, PER_W // SC_LANES, step, 0)
    pltpu.sync_copy(vs, sco_hbm.at[pl.ds(base, PER_W)])
    pltpu.sync_copy(vl, lab_hbm.at[pl.ds(base, PER_W)])


_sc_head = functools.partial(
    pl.kernel,
    mesh=plsc.VectorSubcoreMesh(core_axis_name="c", subcore_axis_name="s"),
    out_type=[
        jax.ShapeDtypeStruct((N_PAD,), jnp.float32),
        jax.ShapeDtypeStruct((N_PAD,), jnp.float32),
    ],
    scratch_types=[
        pltpu.VMEM((PER_W,), jnp.float32),
        pltpu.VMEM((PER_W,), jnp.float32),
        pltpu.VMEM((PER_W,), jnp.float32),
        pltpu.VMEM((PER_W,), jnp.float32),
        pltpu.VMEM((PER_W,), jnp.float32),
    ],
)(_sc_head_body)


def kernel(boxes, cls_preds):
    cp = jnp.pad(cls_preds, ((0, N_PAD - N_BOXES), (0, 0)))
    cpt = cp.T
    scores_p, labels_p = _sc_head(cpt[0], cpt[1], cpt[2])
    scores = scores_p[:N_BOXES]
    labels = labels_p[:N_BOXES].astype(jnp.int32)
    top_scores, top_idx = jax.lax.top_k(scores, N_PRE)
    b = boxes[top_idx]
    lbl = labels[top_idx].astype(jnp.float32)

    planes = jnp.concatenate(
        [
            b[:, 0].reshape(ROWS, 128),
            b[:, 1].reshape(ROWS, 128),
            b[:, 2].reshape(ROWS, 128),
            b[:, 3].reshape(ROWS, 128),
        ],
        axis=0,
    )
    packed = jnp.stack(
        [b[:, 0], b[:, 1], b[:, 2], b[:, 3], top_scores, lbl], axis=0)

    out = pl.pallas_call(
        _nms_scan_body,
        in_specs=[
            pl.BlockSpec(memory_space=pltpu.VMEM),
            pl.BlockSpec(memory_space=pltpu.SMEM),
        ],
        out_shape=jax.ShapeDtypeStruct((OUT_ROWS, 8), jnp.float32),
    )(planes, packed)

    rois = out[:N_POST, 0:4]
    roi_scores = out[:N_POST, 4]
    roi_labels = out[:N_POST, 5].astype(jnp.int32)
    return rois, roi_scores, roi_labels


# speedup vs baseline: 1.2142x; 1.0400x over previous
import functools
import jax
import jax.numpy as jnp
from jax import lax
from jax.experimental import pallas as pl
from jax.experimental.pallas import tpu as pltpu
from jax.experimental.pallas import tpu_sc as plsc

N_BOXES = 20000
NUM_CLASS = 3
N_PRE = 2048
N_POST = 500
THRESH = 0.7
ROWS = N_PRE // 128
OUT_ROWS = 512


def _tree(x, op, axis, size):
    s = 1
    while s < size:
        x = op(x, pltpu.roll(x, s, axis))
        s *= 2
    return x


def _nms_scan_body(planes_ref, packed_ref, scores_ref, out_ref):
    cx = planes_ref[0 * ROWS:1 * ROWS, :]
    cy = planes_ref[1 * ROWS:2 * ROWS, :]
    w = planes_ref[2 * ROWS:3 * ROWS, :]
    h = planes_ref[3 * ROWS:4 * ROWS, :]

    x1 = cx - 0.5 * w
    y1 = cy - 0.5 * h
    x2 = cx + 0.5 * w
    y2 = cy + 0.5 * h
    area = (x2 - x1) * (y2 - y1)

    row_i = jax.lax.broadcasted_iota(jnp.int32, (ROWS, 128), 0)
    col_i = jax.lax.broadcasted_iota(jnp.int32, (ROWS, 128), 1)
    iota2 = (row_i * 128 + col_i).astype(jnp.float32)
    lane128 = jax.lax.broadcasted_iota(jnp.int32, (1, 128), 1)

    def body(i, sup):
        cand = jnp.where(sup != 0, 2.0 * N_PRE, iota2)
        m = jnp.broadcast_to(jnp.min(cand, axis=1, keepdims=True), (ROWS, 128))
        minb = _tree(m, jnp.minimum, 0, ROWS)

        idxf = minb[0, 0]
        valid = idxf < float(N_PRE)
        idxc = jnp.minimum(idxf, float(N_PRE - 1)).astype(jnp.int32)
        cxs = packed_ref[0, idxc]
        cys = packed_ref[1, idxc]
        ws = packed_ref[2, idxc]
        hs = packed_ref[3, idxc]
        lbs = packed_ref[4, idxc]
        scs = scores_ref[idxc]
        x1s = cxs - 0.5 * ws
        y1s = cys - 0.5 * hs
        x2s = cxs + 0.5 * ws
        y2s = cys + 0.5 * hs
        area_s = (x2s - x1s) * (y2s - y1s)

        iw = jnp.clip(jnp.minimum(x2, x2s) - jnp.maximum(x1, x1s), 0.0, None)
        ih = jnp.clip(jnp.minimum(y2, y2s) - jnp.maximum(y1, y1s), 0.0, None)
        inter = iw * ih
        iou = inter / (area + area_s - inter + 1e-8)
        supn = jnp.where(valid, sup | (iou >= THRESH).astype(jnp.int32), sup)

        vf = jnp.where(valid, 1.0, 0.0)
        row = (jnp.where(lane128 == 0, cxs, 0.0)
               + jnp.where(lane128 == 1, cys, 0.0)
               + jnp.where(lane128 == 2, ws, 0.0)
               + jnp.where(lane128 == 3, hs, 0.0)
               + jnp.where(lane128 == 4, scs, 0.0)
               + jnp.where(lane128 == 5, lbs + 1.0, 0.0)) * vf
        out_ref[pl.ds(i, 1), :] = row[:, 0:8]
        return supn

    sup0 = jnp.zeros((ROWS, 128), dtype=jnp.int32)
    jax.lax.fori_loop(0, N_POST, body, sup0)


N_PAD = 20480
PER_W = N_PAD // 32
SC_LANES = 16


def _sc_head_body(c_hbm, sco_hbm, lab_hbm, v0, v1, v2, vs, vl):
    wid = lax.axis_index("s") * 2 + lax.axis_index("c")
    base = wid * PER_W
    pltpu.sync_copy(c_hbm.at[pl.ds(0 * N_PAD + base, PER_W)], v0)
    pltpu.sync_copy(c_hbm.at[pl.ds(1 * N_PAD + base, PER_W)], v1)
    pltpu.sync_copy(c_hbm.at[pl.ds(2 * N_PAD + base, PER_W)], v2)

    def step(i, _):
        s0 = v0[pl.ds(i * SC_LANES, SC_LANES)]
        s1 = v1[pl.ds(i * SC_LANES, SC_LANES)]
        s2 = v2[pl.ds(i * SC_LANES, SC_LANES)]
        m = jnp.maximum(jnp.maximum(s0, s1), s2)
        lab = jnp.where(s0 == m, 0.0, jnp.where(s1 == m, 1.0, 2.0))
        vs[pl.ds(i * SC_LANES, SC_LANES)] = m
        vl[pl.ds(i * SC_LANES, SC_LANES)] = lab
        return 0

    lax.fori_loop(0, PER_W // SC_LANES, step, 0)
    pltpu.sync_copy(vs, sco_hbm.at[pl.ds(base, PER_W)])
    pltpu.sync_copy(vl, lab_hbm.at[pl.ds(base, PER_W)])


_sc_head = functools.partial(
    pl.kernel,
    mesh=plsc.VectorSubcoreMesh(core_axis_name="c", subcore_axis_name="s"),
    out_type=[
        jax.ShapeDtypeStruct((N_PAD,), jnp.float32),
        jax.ShapeDtypeStruct((N_PAD,), jnp.float32),
    ],
    scratch_types=[
        pltpu.VMEM((PER_W,), jnp.float32),
        pltpu.VMEM((PER_W,), jnp.float32),
        pltpu.VMEM((PER_W,), jnp.float32),
        pltpu.VMEM((PER_W,), jnp.float32),
        pltpu.VMEM((PER_W,), jnp.float32),
    ],
)(_sc_head_body)


def kernel(boxes, cls_preds):
    cpt = jnp.pad(cls_preds, ((0, N_PAD - N_BOXES), (0, 0))).T.reshape(-1)
    scores_p, labels_p = _sc_head(cpt)
    scores = scores_p[:N_BOXES]
    top_scores, top_idx = jax.lax.top_k(scores, N_PRE)
    comb = jnp.concatenate([boxes, labels_p[:N_BOXES, None]], axis=1)
    gt = comb[top_idx].T
    planes = gt[0:4].reshape(4 * ROWS, 128)

    out = pl.pallas_call(
        _nms_scan_body,
        in_specs=[
            pl.BlockSpec(memory_space=pltpu.VMEM),
            pl.BlockSpec(memory_space=pltpu.SMEM),
            pl.BlockSpec(memory_space=pltpu.SMEM),
        ],
        out_shape=jax.ShapeDtypeStruct((OUT_ROWS, 8), jnp.float32),
    )(planes, gt, top_scores)

    rois = out[:N_POST, 0:4]
    roi_scores = out[:N_POST, 4]
    roi_labels = out[:N_POST, 5].astype(jnp.int32)
    return rois, roi_scores, roi_labels
